# baseline (speedup 1.0000x reference)
import functools

import jax
import jax.numpy as jnp
from jax import lax
from jax.experimental import pallas as pl
from jax.experimental.pallas import tpu as pltpu
from jax.experimental.pallas import tpu_sc as plsc

B = 16384
NG = 200
D = 32
NC = 2
NS = 16
NW = NC * NS
ROWS_PER_TILE = B // NW
IDX_BLOCK = 64
N_BLOCKS = ROWS_PER_TILE // IDX_BLOCK
L = 16
RED_UNROLL = 8
NGP = 256


def _reduce_bag(rows_v, r):
    zero = jnp.zeros((L,), jnp.float32)

    def body(it, carry):
        a = list(carry)
        j0 = it * RED_UNROLL
        for c in range(RED_UNROLL):
            lo = rows_v[j0 + c, 0:L]
            hi = rows_v[j0 + c, L:D]
            k = (c % 4) * 2
            a[k] = a[k] + lo
            a[k + 1] = a[k + 1] + hi
        return tuple(a)

    acc = lax.fori_loop(0, NG // RED_UNROLL, body, (zero,) * 8)
    lo = (acc[0] + acc[2]) + (acc[4] + acc[6])
    hi = (acc[1] + acc[3]) + (acc[5] + acc[7])
    return lo, hi


def _sc_bag_sums(idxs, emb):
    mesh = plsc.VectorSubcoreMesh(core_axis_name="c", subcore_axis_name="s")

    @functools.partial(
        pl.kernel,
        out_type=jax.ShapeDtypeStruct((B, D), jnp.float32),
        mesh=mesh,
        scratch_types=[
            pltpu.VMEM((IDX_BLOCK, NGP), jnp.int32),
            pltpu.VMEM((NG, D), jnp.float32),
            pltpu.VMEM((NG, D), jnp.float32),
            pltpu.VMEM((ROWS_PER_TILE, D), jnp.float32),
            pltpu.SemaphoreType.DMA,
            pltpu.SemaphoreType.DMA,
        ],
        compiler_params=pltpu.CompilerParams(use_tc_tiling_on_sc=False),
    )
    def k(idx_hbm, emb_hbm, out_hbm, idx_v, rows0, rows1, out_v, sem0, sem1):
        wid = lax.axis_index("s") * NC + lax.axis_index("c")
        base = wid * ROWS_PER_TILE
        rows = (rows0, rows1)
        sems = (sem0, sem1)

        def gather(r, buf):
            pltpu.async_copy(
                emb_hbm.at[idx_v.at[r, pl.ds(0, NG)]], rows[buf], sems[buf])

        def wait(buf):
            pltpu.make_async_copy(
                emb_hbm.at[idx_v.at[0, pl.ds(0, NG)]], rows[buf], sems[buf]
            ).wait()

        for blk in range(N_BLOCKS):
            pltpu.sync_copy(
                idx_hbm.at[pl.ds(base + blk * IDX_BLOCK, IDX_BLOCK)],
                idx_v)
            gather(0, 0)
            gather(1, 1)

            def step(i, carry):
                for b in range(2):
                    r = i + b
                    wait(b)
                    lo, hi = _reduce_bag(rows[b], 0)
                    orow = blk * IDX_BLOCK + r
                    out_v[orow, 0:L] = lo
                    out_v[orow, L:D] = hi

                    @pl.when(r + 2 < IDX_BLOCK)
                    def _prefetch(b=b, r=r):
                        gather(r + 2, b)
                return carry

            lax.fori_loop(0, IDX_BLOCK // 2, lambda i, c: step(i * 2, c), 0)

        pltpu.sync_copy(out_v, out_hbm.at[pl.ds(base, ROWS_PER_TILE)])

    idxs_p = jnp.pad(idxs, ((0, 0), (0, NGP - NG)))
    return k(idxs_p, emb)


def _normalize_block(x_ref, o_ref):
    x = x_ref[...]
    norm = jnp.sqrt(jnp.sum(x * x, axis=1, keepdims=True))
    o_ref[...] = x / jnp.maximum(norm, 1e-12)


def _tc_normalize(vecs):
    blk = 2048
    return pl.pallas_call(
        _normalize_block,
        out_shape=jax.ShapeDtypeStruct((B, D), jnp.float32),
        grid=(B // blk,),
        in_specs=[pl.BlockSpec((blk, D), lambda i: (i, 0))],
        out_specs=pl.BlockSpec((blk, D), lambda i: (i, 0)),
    )(vecs)


def kernel(idxs, emb):
    return _tc_normalize(_sc_bag_sums(idxs, emb))


# speedup vs baseline: 1.1698x; 1.1698x over previous
import functools

import jax
import jax.numpy as jnp
from jax import lax
from jax.experimental import pallas as pl
from jax.experimental.pallas import tpu as pltpu
from jax.experimental.pallas import tpu_sc as plsc

B = 16384
NG = 200
D = 32
NC = 2
NS = 16
NW = NC * NS
ROWS_PER_TILE = B // NW
IDX_BLOCK = 64
N_BLOCKS = ROWS_PER_TILE // IDX_BLOCK
L = 16
RED_UNROLL = 8
NGP = 256
NBUF = 4


def _reduce_bag(rows_v, r):
    zero = jnp.zeros((L,), jnp.float32)

    def body(it, carry):
        a = list(carry)
        j0 = it * RED_UNROLL
        for c in range(RED_UNROLL):
            lo = rows_v[j0 + c, 0:L]
            hi = rows_v[j0 + c, L:D]
            k = (c % 4) * 2
            a[k] = a[k] + lo
            a[k + 1] = a[k + 1] + hi
        return tuple(a)

    acc = lax.fori_loop(0, NG // RED_UNROLL, body, (zero,) * 8)
    lo = (acc[0] + acc[2]) + (acc[4] + acc[6])
    hi = (acc[1] + acc[3]) + (acc[5] + acc[7])
    return lo, hi


def _sc_bag_sums(idxs, emb):
    mesh = plsc.VectorSubcoreMesh(core_axis_name="c", subcore_axis_name="s")

    @functools.partial(
        pl.kernel,
        out_type=jax.ShapeDtypeStruct((B, D), jnp.float32),
        mesh=mesh,
        scratch_types=[
            pltpu.VMEM((IDX_BLOCK, NGP), jnp.int32),
            pltpu.VMEM((NG, D), jnp.float32),
            pltpu.VMEM((NG, D), jnp.float32),
            pltpu.VMEM((NG, D), jnp.float32),
            pltpu.VMEM((NG, D), jnp.float32),
            pltpu.VMEM((ROWS_PER_TILE, D), jnp.float32),
            pltpu.SemaphoreType.DMA,
            pltpu.SemaphoreType.DMA,
            pltpu.SemaphoreType.DMA,
            pltpu.SemaphoreType.DMA,
        ],
        compiler_params=pltpu.CompilerParams(use_tc_tiling_on_sc=False),
    )
    def k(idx_hbm, emb_hbm, out_hbm, idx_v, rows0, rows1, rows2, rows3,
          out_v, sem0, sem1, sem2, sem3):
        wid = lax.axis_index("s") * NC + lax.axis_index("c")
        base = wid * ROWS_PER_TILE
        rows = (rows0, rows1, rows2, rows3)
        sems = (sem0, sem1, sem2, sem3)

        def gather(r, buf):
            pltpu.async_copy(
                emb_hbm.at[idx_v.at[r, pl.ds(0, NG)]], rows[buf], sems[buf])

        def wait(buf):
            pltpu.make_async_copy(
                emb_hbm.at[idx_v.at[0, pl.ds(0, NG)]], rows[buf], sems[buf]
            ).wait()

        for blk in range(N_BLOCKS):
            pltpu.sync_copy(
                idx_hbm.at[pl.ds(base + blk * IDX_BLOCK, IDX_BLOCK)],
                idx_v)
            gather(0, 0)
            gather(1, 1)
            gather(2, 2)
            gather(3, 3)

            def step(i, carry):
                for b in range(NBUF):
                    r = i + b
                    wait(b)
                    lo, hi = _reduce_bag(rows[b], 0)
                    orow = blk * IDX_BLOCK + r
                    out_v[orow, 0:L] = lo
                    out_v[orow, L:D] = hi

                    @pl.when(r + NBUF < IDX_BLOCK)
                    def _prefetch(b=b, r=r):
                        gather(r + NBUF, b)
                return carry

            lax.fori_loop(0, IDX_BLOCK // NBUF,
                          lambda i, c: step(i * NBUF, c), 0)

        pltpu.sync_copy(out_v, out_hbm.at[pl.ds(base, ROWS_PER_TILE)])

    idxs_p = jnp.pad(idxs, ((0, 0), (0, NGP - NG)))
    return k(idxs_p, emb)


def _normalize_block(x_ref, o_ref):
    x = x_ref[...]
    norm = jnp.sqrt(jnp.sum(x * x, axis=1, keepdims=True))
    o_ref[...] = x / jnp.maximum(norm, 1e-12)


def _tc_normalize(vecs):
    blk = 2048
    return pl.pallas_call(
        _normalize_block,
        out_shape=jax.ShapeDtypeStruct((B, D), jnp.float32),
        grid=(B // blk,),
        in_specs=[pl.BlockSpec((blk, D), lambda i: (i, 0))],
        out_specs=pl.BlockSpec((blk, D), lambda i: (i, 0)),
    )(vecs)


def kernel(idxs, emb):
    return _tc_normalize(_sc_bag_sums(idxs, emb))


# speedup vs baseline: 1.2106x; 1.0348x over previous
import functools

import jax
import jax.numpy as jnp
from jax import lax
from jax.experimental import pallas as pl
from jax.experimental.pallas import tpu as pltpu
from jax.experimental.pallas import tpu_sc as plsc

B = 16384
NG = 200
D = 32
NC = 2
NS = 16
NW = NC * NS
ROWS_PER_TILE = B // NW
IDX_BLOCK = 64
N_BLOCKS = ROWS_PER_TILE // IDX_BLOCK
L = 16
RED_UNROLL = 8
NGP = 256
NBUF = 8


def _reduce_bag(rows_v, r):
    zero = jnp.zeros((L,), jnp.float32)

    def body(it, carry):
        a = list(carry)
        j0 = it * RED_UNROLL
        for c in range(RED_UNROLL):
            lo = rows_v[j0 + c, 0:L]
            hi = rows_v[j0 + c, L:D]
            k = (c % 4) * 2
            a[k] = a[k] + lo
            a[k + 1] = a[k + 1] + hi
        return tuple(a)

    acc = lax.fori_loop(0, NG // RED_UNROLL, body, (zero,) * 8)
    lo = (acc[0] + acc[2]) + (acc[4] + acc[6])
    hi = (acc[1] + acc[3]) + (acc[5] + acc[7])
    return lo, hi


def _sc_bag_sums(idxs, emb):
    mesh = plsc.VectorSubcoreMesh(core_axis_name="c", subcore_axis_name="s")

    @functools.partial(
        pl.kernel,
        out_type=jax.ShapeDtypeStruct((B, D), jnp.float32),
        mesh=mesh,
        scratch_types=(
            [pltpu.VMEM((IDX_BLOCK, NGP), jnp.int32)]
            + [pltpu.VMEM((NG, D), jnp.float32) for _ in range(NBUF)]
            + [pltpu.VMEM((ROWS_PER_TILE, D), jnp.float32)]
            + [pltpu.SemaphoreType.DMA for _ in range(NBUF)]
        ),
        compiler_params=pltpu.CompilerParams(use_tc_tiling_on_sc=False),
    )
    def k(idx_hbm, emb_hbm, out_hbm, idx_v, *scratch):
        rows = scratch[:NBUF]
        out_v = scratch[NBUF]
        sems = scratch[NBUF + 1:]
        wid = lax.axis_index("s") * NC + lax.axis_index("c")
        base = wid * ROWS_PER_TILE

        def gather(r, buf):
            pltpu.async_copy(
                emb_hbm.at[idx_v.at[r, pl.ds(0, NG)]], rows[buf], sems[buf])

        def wait(buf):
            pltpu.make_async_copy(
                emb_hbm.at[idx_v.at[0, pl.ds(0, NG)]], rows[buf], sems[buf]
            ).wait()

        for blk in range(N_BLOCKS):
            pltpu.sync_copy(
                idx_hbm.at[pl.ds(base + blk * IDX_BLOCK, IDX_BLOCK)],
                idx_v)
            for b in range(NBUF):
                gather(b, b)

            def step(i, carry):
                for b in range(NBUF):
                    r = i + b
                    wait(b)
                    lo, hi = _reduce_bag(rows[b], 0)
                    orow = blk * IDX_BLOCK + r
                    out_v[orow, 0:L] = lo
                    out_v[orow, L:D] = hi

                    @pl.when(r + NBUF < IDX_BLOCK)
                    def _prefetch(b=b, r=r):
                        gather(r + NBUF, b)
                return carry

            lax.fori_loop(0, IDX_BLOCK // NBUF,
                          lambda i, c: step(i * NBUF, c), 0)

        pltpu.sync_copy(out_v, out_hbm.at[pl.ds(base, ROWS_PER_TILE)])

    idxs_p = jnp.pad(idxs, ((0, 0), (0, NGP - NG)))
    return k(idxs_p, emb)


def _normalize_block(x_ref, o_ref):
    x = x_ref[...]
    norm = jnp.sqrt(jnp.sum(x * x, axis=1, keepdims=True))
    o_ref[...] = x / jnp.maximum(norm, 1e-12)


def _tc_normalize(vecs):
    blk = 2048
    return pl.pallas_call(
        _normalize_block,
        out_shape=jax.ShapeDtypeStruct((B, D), jnp.float32),
        grid=(B // blk,),
        in_specs=[pl.BlockSpec((blk, D), lambda i: (i, 0))],
        out_specs=pl.BlockSpec((blk, D), lambda i: (i, 0)),
    )(vecs)


def kernel(idxs, emb):
    return _tc_normalize(_sc_bag_sums(idxs, emb))


# speedup vs baseline: 1.2232x; 1.0104x over previous
import functools

import jax
import jax.numpy as jnp
from jax import lax
from jax.experimental import pallas as pl
from jax.experimental.pallas import tpu as pltpu
from jax.experimental.pallas import tpu_sc as plsc

B = 16384
NG = 200
D = 32
NC = 2
NS = 16
NW = NC * NS
ROWS_PER_TILE = B // NW
IDX_BLOCK = 64
N_BLOCKS = ROWS_PER_TILE // IDX_BLOCK
L = 16
RED_UNROLL = 8
NGP = 256
NBUF = 8


def _reduce_bag(rows_v, r):
    zero = jnp.zeros((L,), jnp.float32)

    def body(it, carry):
        a = list(carry)
        j0 = it * RED_UNROLL
        for c in range(RED_UNROLL):
            lo = rows_v[j0 + c, 0:L]
            hi = rows_v[j0 + c, L:D]
            k = (c % 4) * 2
            a[k] = a[k] + lo
            a[k + 1] = a[k + 1] + hi
        return tuple(a)

    acc = lax.fori_loop(0, NG // RED_UNROLL, body, (zero,) * 8)
    lo = (acc[0] + acc[2]) + (acc[4] + acc[6])
    hi = (acc[1] + acc[3]) + (acc[5] + acc[7])
    return lo, hi


def _sc_bag_sums(idxs, emb):
    mesh = plsc.VectorSubcoreMesh(core_axis_name="c", subcore_axis_name="s")

    @functools.partial(
        pl.kernel,
        out_type=jax.ShapeDtypeStruct((B, D), jnp.float32),
        mesh=mesh,
        scratch_types=(
            [pltpu.VMEM((IDX_BLOCK, NGP), jnp.int32) for _ in range(2)]
            + [pltpu.VMEM((NG, D), jnp.float32) for _ in range(NBUF)]
            + [pltpu.VMEM((ROWS_PER_TILE, D), jnp.float32)]
            + [pltpu.SemaphoreType.DMA for _ in range(NBUF + 2)]
        ),
        compiler_params=pltpu.CompilerParams(use_tc_tiling_on_sc=False),
    )
    def k(idx_hbm, emb_hbm, out_hbm, *scratch):
        idx_bufs = scratch[:2]
        rows = scratch[2:2 + NBUF]
        out_v = scratch[2 + NBUF]
        sems = scratch[3 + NBUF:3 + 2 * NBUF]
        idx_sems = scratch[3 + 2 * NBUF:]
        wid = lax.axis_index("s") * NC + lax.axis_index("c")
        base = wid * ROWS_PER_TILE

        def stage_idx(blk):
            pltpu.async_copy(
                idx_hbm.at[pl.ds(base + blk * IDX_BLOCK, IDX_BLOCK)],
                idx_bufs[blk % 2], idx_sems[blk % 2])

        def wait_idx(blk):
            pltpu.make_async_copy(
                idx_hbm.at[pl.ds(base + blk * IDX_BLOCK, IDX_BLOCK)],
                idx_bufs[blk % 2], idx_sems[blk % 2]).wait()

        def gather(idx_v, r, buf):
            pltpu.async_copy(
                emb_hbm.at[idx_v.at[r, pl.ds(0, NG)]], rows[buf], sems[buf])

        def wait(idx_v, buf):
            pltpu.make_async_copy(
                emb_hbm.at[idx_v.at[0, pl.ds(0, NG)]], rows[buf], sems[buf]
            ).wait()

        stage_idx(0)
        wait_idx(0)
        for blk in range(N_BLOCKS):
            idx_v = idx_bufs[blk % 2]
            if blk + 1 < N_BLOCKS:
                stage_idx(blk + 1)
            for b in range(NBUF):
                gather(idx_v, b, b)

            def step(i, carry, idx_v=idx_v, blk=blk):
                for b in range(NBUF):
                    r = i + b
                    wait(idx_v, b)
                    lo, hi = _reduce_bag(rows[b], 0)
                    orow = blk * IDX_BLOCK + r
                    out_v[orow, 0:L] = lo
                    out_v[orow, L:D] = hi

                    @pl.when(r + NBUF < IDX_BLOCK)
                    def _prefetch(b=b, r=r):
                        gather(idx_v, r + NBUF, b)
                return carry

            lax.fori_loop(0, IDX_BLOCK // NBUF,
                          lambda i, c, s=step: s(i * NBUF, c), 0)
            if blk + 1 < N_BLOCKS:
                wait_idx(blk + 1)

        pltpu.sync_copy(out_v, out_hbm.at[pl.ds(base, ROWS_PER_TILE)])

    idxs_p = jnp.pad(idxs, ((0, 0), (0, NGP - NG)))
    return k(idxs_p, emb)


def _normalize_block(x_ref, o_ref):
    x = x_ref[...]
    norm = jnp.sqrt(jnp.sum(x * x, axis=1, keepdims=True))
    o_ref[...] = x / jnp.maximum(norm, 1e-12)


def _tc_normalize(vecs):
    blk = 2048
    return pl.pallas_call(
        _normalize_block,
        out_shape=jax.ShapeDtypeStruct((B, D), jnp.float32),
        grid=(B // blk,),
        in_specs=[pl.BlockSpec((blk, D), lambda i: (i, 0))],
        out_specs=pl.BlockSpec((blk, D), lambda i: (i, 0)),
    )(vecs)


def kernel(idxs, emb):
    return _tc_normalize(_sc_bag_sums(idxs, emb))


# speedup vs baseline: 1.2368x; 1.0112x over previous
import functools

import jax
import jax.numpy as jnp
from jax import lax
from jax.experimental import pallas as pl
from jax.experimental.pallas import tpu as pltpu
from jax.experimental.pallas import tpu_sc as plsc

B = 16384
NG = 200
D = 32
NC = 2
NS = 16
NW = NC * NS
ROWS_PER_TILE = B // NW
IDX_BLOCK = 64
N_BLOCKS = ROWS_PER_TILE // IDX_BLOCK
L = 16
RED_UNROLL = 8
NGP = 256
NBUF = 8


def _reduce_bag(rows_v, r):
    zero = jnp.zeros((L,), jnp.float32)

    def body(it, carry):
        a = list(carry)
        j0 = it * RED_UNROLL
        for c in range(RED_UNROLL):
            lo = rows_v[j0 + c, 0:L]
            hi = rows_v[j0 + c, L:D]
            k = (c % 4) * 2
            a[k] = a[k] + lo
            a[k + 1] = a[k + 1] + hi
        return tuple(a)

    acc = lax.fori_loop(0, NG // RED_UNROLL, body, (zero,) * 8)
    lo = (acc[0] + acc[2]) + (acc[4] + acc[6])
    hi = (acc[1] + acc[3]) + (acc[5] + acc[7])
    return lo, hi


def _sc_bag_sums(idxs, emb):
    mesh = plsc.VectorSubcoreMesh(core_axis_name="c", subcore_axis_name="s")

    @functools.partial(
        pl.kernel,
        out_type=jax.ShapeDtypeStruct((B, D), jnp.float32),
        mesh=mesh,
        scratch_types=(
            [pltpu.VMEM((IDX_BLOCK * NG,), jnp.int32) for _ in range(2)]
            + [pltpu.VMEM((NG, D), jnp.float32) for _ in range(NBUF)]
            + [pltpu.VMEM((ROWS_PER_TILE, D), jnp.float32)]
            + [pltpu.SemaphoreType.DMA for _ in range(NBUF + 2)]
        ),
        compiler_params=pltpu.CompilerParams(use_tc_tiling_on_sc=False),
    )
    def k(idx_hbm, emb_hbm, out_hbm, *scratch):
        idx_bufs = scratch[:2]
        rows = scratch[2:2 + NBUF]
        out_v = scratch[2 + NBUF]
        sems = scratch[3 + NBUF:3 + 2 * NBUF]
        idx_sems = scratch[3 + 2 * NBUF:]
        wid = lax.axis_index("s") * NC + lax.axis_index("c")
        base = wid * ROWS_PER_TILE

        def stage_idx(blk):
            pltpu.async_copy(
                idx_hbm.at[pl.ds((base + blk * IDX_BLOCK) * NG,
                                 IDX_BLOCK * NG)],
                idx_bufs[blk % 2], idx_sems[blk % 2])

        def wait_idx(blk):
            pltpu.make_async_copy(
                idx_hbm.at[pl.ds((base + blk * IDX_BLOCK) * NG,
                                 IDX_BLOCK * NG)],
                idx_bufs[blk % 2], idx_sems[blk % 2]).wait()

        def gather(idx_v, r, buf):
            pltpu.async_copy(
                emb_hbm.at[idx_v.at[pl.ds(r * NG, NG)]], rows[buf], sems[buf])

        def wait(idx_v, buf):
            pltpu.make_async_copy(
                emb_hbm.at[idx_v.at[pl.ds(0, NG)]], rows[buf], sems[buf]
            ).wait()

        stage_idx(0)
        wait_idx(0)
        for blk in range(N_BLOCKS):
            idx_v = idx_bufs[blk % 2]
            if blk + 1 < N_BLOCKS:
                stage_idx(blk + 1)
            for b in range(NBUF):
                gather(idx_v, b, b)

            def step(i, carry, idx_v=idx_v, blk=blk):
                for b in range(NBUF):
                    r = i + b
                    wait(idx_v, b)
                    lo, hi = _reduce_bag(rows[b], 0)
                    orow = blk * IDX_BLOCK + r
                    out_v[orow, 0:L] = lo
                    out_v[orow, L:D] = hi

                    @pl.when(r + NBUF < IDX_BLOCK)
                    def _prefetch(b=b, r=r):
                        gather(idx_v, r + NBUF, b)
                return carry

            lax.fori_loop(0, IDX_BLOCK // NBUF,
                          lambda i, c, s=step: s(i * NBUF, c), 0)
            if blk + 1 < N_BLOCKS:
                wait_idx(blk + 1)

        pltpu.sync_copy(out_v, out_hbm.at[pl.ds(base, ROWS_PER_TILE)])

    return k(idxs.reshape(-1), emb)


def _normalize_block(x_ref, o_ref):
    x = x_ref[...]
    norm = jnp.sqrt(jnp.sum(x * x, axis=1, keepdims=True))
    o_ref[...] = x / jnp.maximum(norm, 1e-12)


def _tc_normalize(vecs):
    blk = 2048
    return pl.pallas_call(
        _normalize_block,
        out_shape=jax.ShapeDtypeStruct((B, D), jnp.float32),
        grid=(B // blk,),
        in_specs=[pl.BlockSpec((blk, D), lambda i: (i, 0))],
        out_specs=pl.BlockSpec((blk, D), lambda i: (i, 0)),
    )(vecs)


def kernel(idxs, emb):
    return _tc_normalize(_sc_bag_sums(idxs, emb))
